# baseline (speedup 1.0000x reference)
import functools

import jax
import jax.numpy as jnp
from jax.experimental import pallas as pl
from jax.experimental.pallas import tpu as pltpu
from jax.experimental.pallas import tpu_sc as plsc


def kernel(x, pos_embedding):
    B, S, D = x.shape
    W = 4
    mesh = plsc.VectorSubcoreMesh(core_axis_name="core", subcore_axis_name="subcore")

    @functools.partial(
        pl.kernel,
        out_type=jax.ShapeDtypeStruct((B, S, D), x.dtype),
        mesh=mesh,
    )
    def sc_k(x_hbm, e_hbm, o_hbm):
        def body(x_v, e_v, o_v):
            for r in range(W):
                for c in range(0, D, 16):
                    o_v.at[0, r, pl.ds(c, 16)][...] = (
                        x_v.at[0, r, pl.ds(c, 16)][...]
                        + e_v.at[r, pl.ds(c, 16)][...]
                    )

        pltpu.emit_pipeline(
            body,
            grid=(S // W, B),
            in_specs=[
                pl.BlockSpec((1, W, D), lambda i, b: (b, i, 0)),
                pl.BlockSpec((W, D), lambda i, b: (i, 0)),
            ],
            out_specs=[pl.BlockSpec((1, W, D), lambda i, b: (b, i, 0))],
            core_axis_name=("core", "subcore"),
            dimension_semantics=(pltpu.PARALLEL, pltpu.ARBITRARY),
        )(x_hbm, e_hbm, o_hbm)

    return sc_k(x, pos_embedding)


# speedup vs baseline: 3.9281x; 3.9281x over previous
import jax
import jax.numpy as jnp
from jax.experimental import pallas as pl


def _add_body(x_ref, emb_ref, o_ref):
    o_ref[...] = x_ref[...] + emb_ref[...]


def kernel(x, pos_embedding):
    B, S, D = x.shape
    R = 1024
    C = S // R
    x2 = x.reshape(B * S, D)
    out = pl.pallas_call(
        _add_body,
        grid=(C, B),
        in_specs=[
            pl.BlockSpec((R, D), lambda c, b: (b * C + c, 0)),
            pl.BlockSpec((R, D), lambda c, b: (c, 0)),
        ],
        out_specs=pl.BlockSpec((R, D), lambda c, b: (b * C + c, 0)),
        out_shape=jax.ShapeDtypeStruct((B * S, D), x.dtype),
    )(x2, pos_embedding)
    return out.reshape(B, S, D)


# speedup vs baseline: 4.0972x; 1.0430x over previous
import jax
import jax.numpy as jnp
from jax.experimental import pallas as pl


def _add_body(x_ref, emb_ref, o_ref):
    o_ref[...] = x_ref[...] + emb_ref[...][None, :, :]


def kernel(x, pos_embedding):
    B, S, D = x.shape
    BS = 512
    grid = (S // BS,)
    return pl.pallas_call(
        _add_body,
        grid=grid,
        in_specs=[
            pl.BlockSpec((B, BS, D), lambda i: (0, i, 0)),
            pl.BlockSpec((BS, D), lambda i: (i, 0)),
        ],
        out_specs=pl.BlockSpec((B, BS, D), lambda i: (0, i, 0)),
        out_shape=jax.ShapeDtypeStruct((B, S, D), x.dtype),
    )(x, pos_embedding)
